# baseline (speedup 1.0000x reference)
import functools

import jax
import jax.numpy as jnp
from jax import lax
from jax.experimental import pallas as pl
from jax.experimental.pallas import tpu as pltpu
from jax.experimental.pallas import tpu_sc as plsc

NC = 2
NS = 16
NW = NC * NS
LANES = 16


@functools.partial(jax.jit, static_argnums=(3, 4))
def _sc_embed(idx_t, tok128, pos_table, blk, D):
    L, B = idx_t.shape
    V, DP = tok128.shape

    mesh = plsc.VectorSubcoreMesh(core_axis_name="c", subcore_axis_name="s")

    @functools.partial(
        pl.kernel,
        out_type=jax.ShapeDtypeStruct((L, D, B), jnp.float32),
        mesh=mesh,
        compiler_params=pltpu.CompilerParams(use_tc_tiling_on_sc=True,
                                             needs_layout_passes=False),
        scratch_types=[
            pltpu.VMEM((L, blk), jnp.int32),
            pltpu.VMEM((L, D), jnp.float32),
            pltpu.VMEM((blk, DP), jnp.float32),
            pltpu.VMEM((blk, DP), jnp.float32),
            pltpu.VMEM((D, blk), jnp.float32),
            pltpu.VMEM((D, blk), jnp.float32),
            pltpu.SemaphoreType.DMA,
            pltpu.SemaphoreType.DMA,
            pltpu.SemaphoreType.DMA,
            pltpu.SemaphoreType.DMA,
        ],
    )
    def body(idx_hbm, tok_hbm, pos_hbm, out_hbm,
             idx_v, pos_v, rows0, rows1, tr0, tr1,
             sg0, sg1, so0, so1):
        c = lax.axis_index("c")
        s = lax.axis_index("s")
        wid = s * NC + c
        b0 = wid * blk

        pltpu.sync_copy(idx_hbm.at[:, pl.ds(b0, blk)], idx_v)
        pltpu.sync_copy(pos_hbm, pos_v)

        rows = (rows0, rows1)
        trs = (tr0, tr1)
        sgs = (sg0, sg1)
        sos = (so0, so1)

        def fire_gather(p, buf):
            pltpu.async_copy(tok_hbm.at[idx_v.at[p]], rows[buf], sgs[buf])

        fire_gather(0, 0)

        def do_pos(p, buf):
            pltpu.make_async_copy(tok_hbm.at[idx_v.at[p]], rows[buf],
                                  sgs[buf]).wait()

            @pl.when(p + 1 < L)
            def _():
                fire_gather(p + 1, 1 - buf)

            @pl.when(p >= 2)
            def _():
                pltpu.make_async_copy(
                    trs[buf], out_hbm.at[p - 2, :, pl.ds(b0, blk)],
                    sos[buf]).wait()

            rbuf = rows[buf]
            tbuf = trs[buf]
            nq = D // LANES
            pvs = [pos_v[p, pl.ds(q * LANES, LANES)] for q in range(nq)]

            def add_row(b, carry):
                cols = jnp.full((LANES,), b, jnp.int32)
                for q in range(nq):
                    x = rbuf[b, pl.ds(q * LANES, LANES)] + carry[q]
                    rows_i = lax.iota(jnp.int32, LANES) + q * LANES
                    plsc.store_scatter(tbuf, [rows_i, cols], x)
                return carry

            lax.fori_loop(0, blk, add_row, pvs)

            pltpu.async_copy(tbuf, out_hbm.at[p, :, pl.ds(b0, blk)],
                             sos[buf])

        def step2(pp, carry):
            do_pos(pp * 2, 0)
            do_pos(pp * 2 + 1, 1)
            return carry

        lax.fori_loop(0, L // 2, step2, 0)

        for buf in range(2):
            p = L - 2 + buf
            pltpu.make_async_copy(trs[buf], out_hbm.at[p, :, pl.ds(b0, blk)],
                                  sos[buf]).wait()

    return body(idx_t, tok128, pos_table)


def kernel(inputs, token_table, pos_table):
    B, L = inputs.shape
    V, D = token_table.shape
    idx_t = jnp.transpose(inputs).astype(jnp.int32)
    tok128 = jnp.pad(token_table, ((0, 0), (0, 128 - D)))
    out_phys = _sc_embed(idx_t, tok128, pos_table, B // NW, D)
    return jnp.transpose(out_phys, (2, 0, 1))


# speedup vs baseline: 1.0128x; 1.0128x over previous
import functools

import jax
import jax.numpy as jnp
from jax import lax
from jax.experimental import pallas as pl
from jax.experimental.pallas import tpu as pltpu
from jax.experimental.pallas import tpu_sc as plsc

NC = 2
NS = 16
NW = NC * NS
LANES = 16


@functools.partial(jax.jit, static_argnums=(3, 4))
def _sc_embed(idx_t, tok128, pos_table, blk, D):
    L, B = idx_t.shape
    V, DP = tok128.shape

    mesh = plsc.VectorSubcoreMesh(core_axis_name="c", subcore_axis_name="s")

    @functools.partial(
        pl.kernel,
        out_type=jax.ShapeDtypeStruct((L, D, B), jnp.float32),
        mesh=mesh,
        compiler_params=pltpu.CompilerParams(use_tc_tiling_on_sc=True,
                                             needs_layout_passes=False),
        scratch_types=[
            pltpu.VMEM((L, blk), jnp.int32),
            pltpu.VMEM((L, D), jnp.float32),
            pltpu.VMEM((blk, DP), jnp.float32),
            pltpu.VMEM((blk, DP), jnp.float32),
            pltpu.VMEM((D, blk), jnp.float32),
            pltpu.VMEM((D, blk), jnp.float32),
            pltpu.SemaphoreType.DMA,
            pltpu.SemaphoreType.DMA,
            pltpu.SemaphoreType.DMA,
            pltpu.SemaphoreType.DMA,
        ],
    )
    def body(idx_hbm, tok_hbm, pos_hbm, out_hbm,
             idx_v, pos_v, rows0, rows1, tr0, tr1,
             sg0, sg1, so0, so1):
        c = lax.axis_index("c")
        s = lax.axis_index("s")
        wid = s * NC + c
        b0 = wid * blk

        pltpu.sync_copy(idx_hbm.at[:, pl.ds(b0, blk)], idx_v)
        pltpu.sync_copy(pos_hbm, pos_v)

        rows = (rows0, rows1)
        trs = (tr0, tr1)
        sgs = (sg0, sg1)
        sos = (so0, so1)

        def fire_gather(p, buf):
            pltpu.async_copy(tok_hbm.at[idx_v.at[p]], rows[buf], sgs[buf])

        fire_gather(0, 0)

        def do_pos(p, buf):
            pltpu.make_async_copy(tok_hbm.at[idx_v.at[p]], rows[buf],
                                  sgs[buf]).wait()

            @pl.when(p + 1 < L)
            def _():
                fire_gather(p + 1, 1 - buf)

            @pl.when(p >= 2)
            def _():
                pltpu.make_async_copy(
                    trs[buf], out_hbm.at[p - 2, :, pl.ds(b0, blk)],
                    sos[buf]).wait()

            rbuf = rows[buf]
            tbuf = trs[buf]
            nq = D // LANES
            pvs = tuple(pos_v[p, pl.ds(q * LANES, LANES)] for q in range(nq))
            rows_q = tuple(lax.iota(jnp.int32, LANES) + q * LANES
                           for q in range(nq))

            unroll = 8

            def add_rows(ob, carry):
                pv = carry
                base = ob * unroll
                for k in range(unroll):
                    b = base + k
                    cols = jnp.full((LANES,), b, jnp.int32)
                    for q in range(nq):
                        x = rbuf[b, pl.ds(q * LANES, LANES)] + pv[q]
                        plsc.store_scatter(tbuf, [rows_q[q], cols], x)
                return carry

            lax.fori_loop(0, blk // unroll, add_rows, pvs)

            pltpu.async_copy(tbuf, out_hbm.at[p, :, pl.ds(b0, blk)],
                             sos[buf])

        def step2(pp, carry):
            do_pos(pp * 2, 0)
            do_pos(pp * 2 + 1, 1)
            return carry

        lax.fori_loop(0, L // 2, step2, 0)

        for buf in range(2):
            p = L - 2 + buf
            pltpu.make_async_copy(trs[buf], out_hbm.at[p, :, pl.ds(b0, blk)],
                                  sos[buf]).wait()

    return body(idx_t, tok128, pos_table)


def kernel(inputs, token_table, pos_table):
    B, L = inputs.shape
    V, D = token_table.shape
    idx_t = jnp.transpose(inputs).astype(jnp.int32)
    tok128 = jnp.pad(token_table, ((0, 0), (0, 128 - D)))
    out_phys = _sc_embed(idx_t, tok128, pos_table, B // NW, D)
    return jnp.transpose(out_phys, (2, 0, 1))


# speedup vs baseline: 1.9444x; 1.9198x over previous
import functools

import jax
import jax.numpy as jnp
from jax import lax
from jax.experimental import pallas as pl
from jax.experimental.pallas import tpu as pltpu
from jax.experimental.pallas import tpu_sc as plsc

NC = 2
NS = 16
NW = NC * NS
LANES = 16


@functools.partial(jax.jit, static_argnums=(3, 4))
def _sc_embed(idx_t, tok128, pos_table, blk, D):
    L, B = idx_t.shape
    V, DP = tok128.shape

    mesh = plsc.VectorSubcoreMesh(core_axis_name="c", subcore_axis_name="s")

    @functools.partial(
        pl.kernel,
        out_type=jax.ShapeDtypeStruct((L, D, B), jnp.float32),
        mesh=mesh,
        compiler_params=pltpu.CompilerParams(use_tc_tiling_on_sc=True,
                                             needs_layout_passes=False),
        scratch_types=[
            pltpu.VMEM((L, blk), jnp.int32),
            pltpu.VMEM((L, D), jnp.float32),
            pltpu.VMEM((blk, DP), jnp.float32),
            pltpu.VMEM((blk, DP), jnp.float32),
            pltpu.VMEM((D, blk), jnp.float32),
            pltpu.VMEM((D, blk), jnp.float32),
            pltpu.SemaphoreType.DMA,
            pltpu.SemaphoreType.DMA,
            pltpu.SemaphoreType.DMA,
            pltpu.SemaphoreType.DMA,
        ],
    )
    def body(idx_hbm, tok_hbm, pos_hbm, out_hbm,
             idx_v, pos_v, rows0, rows1, tr0, tr1,
             sg0, sg1, so0, so1):
        c = lax.axis_index("c")
        s = lax.axis_index("s")
        wid = s * NC + c
        b0 = wid * blk

        pltpu.sync_copy(idx_hbm.at[:, pl.ds(b0, blk)], idx_v)
        pltpu.sync_copy(pos_hbm, pos_v)

        rows = (rows0, rows1)
        trs = (tr0, tr1)
        sgs = (sg0, sg1)
        sos = (so0, so1)

        def fire_gather(p, buf):
            pltpu.async_copy(tok_hbm.at[idx_v.at[p]], rows[buf], sgs[buf])

        fire_gather(0, 0)

        def do_pos(p, buf):
            pltpu.make_async_copy(tok_hbm.at[idx_v.at[p]], rows[buf],
                                  sgs[buf]).wait()

            @pl.when(p + 1 < L)
            def _():
                fire_gather(p + 1, 1 - buf)

            @pl.when(p >= 2)
            def _():
                pltpu.make_async_copy(
                    trs[buf], out_hbm.at[p - 2, :, pl.ds(b0, blk)],
                    sos[buf]).wait()

            rbuf = rows[buf]
            tbuf = trs[buf]
            nq = D // LANES
            pvs = tuple(pos_v[p, pl.ds(q * LANES, LANES)] for q in range(nq))
            lane = lax.iota(jnp.int32, LANES)
            perms = {s: lane ^ s for s in (1, 2, 4, 8)}
            masks = {s: (lane & s) == 0 for s in (1, 2, 4, 8)}

            def do_bchunk(cb, carry):
                bb = cb * LANES
                for q in range(nq):
                    vs = [rbuf[bb + i, pl.ds(q * LANES, LANES)] + carry[q]
                          for i in range(LANES)]
                    for s in (1, 2, 4, 8):
                        pm, mk = perms[s], masks[s]
                        nv = list(vs)
                        for i in range(LANES):
                            if i & s == 0:
                                pr = i | s
                                lo, hi = vs[i], vs[pr]
                                nv[i] = jnp.where(
                                    mk, lo, hi.at[pm].get(mode="promise_in_bounds"))
                                nv[pr] = jnp.where(
                                    mk, lo.at[pm].get(mode="promise_in_bounds"), hi)
                        vs = nv
                    for i in range(LANES):
                        tbuf[q * LANES + i, pl.ds(bb, LANES)] = vs[i]
                return carry

            lax.fori_loop(0, blk // LANES, do_bchunk, pvs)

            pltpu.async_copy(tbuf, out_hbm.at[p, :, pl.ds(b0, blk)],
                             sos[buf])

        def step2(pp, carry):
            do_pos(pp * 2, 0)
            do_pos(pp * 2 + 1, 1)
            return carry

        lax.fori_loop(0, L // 2, step2, 0)

        for buf in range(2):
            p = L - 2 + buf
            pltpu.make_async_copy(trs[buf], out_hbm.at[p, :, pl.ds(b0, blk)],
                                  sos[buf]).wait()

    return body(idx_t, tok128, pos_table)


def kernel(inputs, token_table, pos_table):
    B, L = inputs.shape
    V, D = token_table.shape
    idx_t = jnp.transpose(inputs).astype(jnp.int32)
    tok128 = jnp.pad(token_table, ((0, 0), (0, 128 - D)))
    out_phys = _sc_embed(idx_t, tok128, pos_table, B // NW, D)
    return jnp.transpose(out_phys, (2, 0, 1))
